# baseline (speedup 1.0000x reference)
import jax
import jax.numpy as jnp
from jax import lax
from jax.experimental import pallas as pl

B, P, F, N = 1024, 100, 128, 512


def _body(k0, k1, k2, k3, k4, idx_ref, tab_ref, out_ref):
    out_ref[:, :, 0 * F:1 * F] = k0[...]
    out_ref[:, :, 1 * F:2 * F] = k1[...]
    out_ref[:, :, 2 * F:3 * F] = k2[...]
    out_ref[:, :, 3 * F:4 * F] = k3[...]
    out_ref[:, :, 4 * F:5 * F] = k4[...]
    idx = idx_ref[0, 0, :]
    onehot = (idx[:, None] == lax.broadcasted_iota(jnp.int32, (P, N), 1)
              ).astype(jnp.float32)
    gathered = jnp.dot(onehot, tab_ref[0], preferred_element_type=jnp.float32)
    out_ref[0, :, 5 * F:6 * F] = gathered


def kernel(known_one_hot, unknown_one_hot, known_differ_one_hot,
           workers_qa_turn_one_hot, workers_max_qa_turn_one_hot,
           personal_nodes, final_node_embed):
    idx3 = personal_nodes.reshape(B, 1, P).astype(jnp.int32)
    feat_spec = pl.BlockSpec((1, P, F), lambda b: (b, 0, 0))
    out = pl.pallas_call(
        _body,
        grid=(B,),
        in_specs=[feat_spec, feat_spec, feat_spec, feat_spec, feat_spec,
                  pl.BlockSpec((1, 1, P), lambda b: (b, 0, 0)),
                  pl.BlockSpec((1, N, F), lambda b: (b, 0, 0))],
        out_specs=pl.BlockSpec((1, P, 6 * F), lambda b: (b, 0, 0)),
        out_shape=jax.ShapeDtypeStruct((B, P, 6 * F), jnp.float32),
    )(known_one_hot, unknown_one_hot, known_differ_one_hot,
      workers_qa_turn_one_hot, workers_max_qa_turn_one_hot,
      idx3, final_node_embed)
    return out
